# baseline (speedup 1.0000x reference)
import functools

import jax
import jax.numpy as jnp
from jax.experimental import pallas as pl
from jax.experimental.pallas import tpu as pltpu

N_NODES = 10000


def _fuse_body(xT, embT, gWT, gb, cWT, cb, out):
    xg = jnp.maximum(jnp.dot(xT[...], gWT[...], preferred_element_type=jnp.float32) + gb[...], 0.0)
    xe = jnp.maximum(jnp.dot(embT[...], cWT[...], preferred_element_type=jnp.float32) + cb[...], 0.0)
    out[...] = jnp.concatenate([xg, xe], axis=1)


def _fused_features(x, emb, gW, gb, cW, cb):
    xT = x[0].T
    embT = emb[0].T
    out = pl.pallas_call(
        _fuse_body,
        out_shape=jax.ShapeDtypeStruct((N_NODES, 128), jnp.float32),
    )(xT, embT, gW.T, gb[None, :], cW.T, cb[None, :])
    return out


def _gatv2(x, src, dst, Wl, bl, Wr, br, att, bias):
    N = x.shape[0]
    xl = x @ Wl + bl
    xr = x @ Wr + br
    e = jax.nn.leaky_relu(xl[src] + xr[dst], 0.2)
    alpha = e @ att
    amax = jax.ops.segment_max(alpha, dst, num_segments=N)
    ex = jnp.exp(alpha - amax[dst])
    denom = jax.ops.segment_sum(ex, dst, num_segments=N)
    a = ex / (denom[dst] + 1e-16)
    out = jax.ops.segment_sum(a[:, None] * xl[src], dst, num_segments=N)
    return out + bias


def kernel(x, emb, edge_index, gW, gb, cW, cb, Wl1, bl1, Wr1, br1, att1, bias1, Wl2, bl2, Wr2, br2, att2, bias2, Wl3, bl3, Wr3, br3, att3, bias3, Wl4, bl4, Wr4, br4, att4, bias4):
    fused = _fused_features(x, emb, gW, gb, cW, cb)
    N = fused.shape[0]
    loop = jnp.arange(N, dtype=edge_index.dtype)
    src = jnp.concatenate([edge_index[0], loop])
    dst = jnp.concatenate([edge_index[1], loop])
    f1 = jax.nn.relu(_gatv2(fused, src, dst, Wl1, bl1, Wr1, br1, att1, bias1))
    f2 = jax.nn.relu(_gatv2(f1, src, dst, Wl2, bl2, Wr2, br2, att2, bias2))
    f3 = jax.nn.relu(_gatv2(f2, src, dst, Wl3, bl3, Wr3, br3, att3, bias3))
    f4 = _gatv2(f3, src, dst, Wl4, bl4, Wr4, br4, att4, bias4)
    return jnp.concatenate([f1, f2, f4], axis=1)


# speedup vs baseline: 3.1399x; 3.1399x over previous
import functools

import jax
import jax.numpy as jnp
from jax import lax
from jax.experimental import pallas as pl
from jax.experimental.pallas import tpu as pltpu
from jax.experimental.pallas import tpu_sc as plsc

N_NODES = 10000
NPB = 320
NW = 32
_F32 = jnp.float32
_I32 = jnp.int32


def _fuse_body(xT, embT, gWT, gb, cWT, cb, out):
    xg = jnp.maximum(jnp.dot(xT[...], gWT[...], preferred_element_type=_F32) + gb[...], 0.0)
    xe = jnp.maximum(jnp.dot(embT[...], cWT[...], preferred_element_type=_F32) + cb[...], 0.0)
    out[...] = jnp.concatenate([xg, xe], axis=1)


def _fused_features(x, emb, gW, gb, cW, cb):
    return pl.pallas_call(
        _fuse_body,
        out_shape=jax.ShapeDtypeStruct((N_NODES, 128), _F32),
    )(x[0].T, emb[0].T, gW.T, gb[None, :], cW.T, cb[None, :])


def _proj_body(h, Wl, bl, Wr, br, xl, xr):
    hb = h[...]
    xl[...] = jnp.dot(hb, Wl[...], preferred_element_type=_F32) + bl[...]
    xr[...] = jnp.dot(hb, Wr[...], preferred_element_type=_F32) + br[...]


def _project(h, Wl, bl, Wr, br):
    n, ic = h.shape
    oc = Wl.shape[1]
    bn = 1000
    grid = n // bn
    return pl.pallas_call(
        _proj_body,
        grid=(grid,),
        in_specs=[
            pl.BlockSpec((bn, ic), lambda i: (i, 0)),
            pl.BlockSpec((ic, oc), lambda i: (0, 0)),
            pl.BlockSpec((1, oc), lambda i: (0, 0)),
            pl.BlockSpec((ic, oc), lambda i: (0, 0)),
            pl.BlockSpec((1, oc), lambda i: (0, 0)),
        ],
        out_specs=[
            pl.BlockSpec((bn, oc), lambda i: (i, 0)),
            pl.BlockSpec((bn, oc), lambda i: (i, 0)),
        ],
        out_shape=[
            jax.ShapeDtypeStruct((n, oc), _F32),
            jax.ShapeDtypeStruct((n, oc), _F32),
        ],
    )(h, Wl, bl[None, :], Wr, br[None, :])


def _shuf(v, idx):
    return v.at[idx].get(mode="promise_in_bounds")


def _allreduce(v, op, iota):
    for sh in (8, 4, 2, 1):
        v = op(v, _shuf(v, jnp.bitwise_xor(iota, sh)))
    return v


def _lane_sums(vecs, iota):
    for sh in (1, 2, 4, 8):
        nxt = []
        for i in range(0, len(vecs), 2):
            a, b = vecs[i], vecs[i + 1]
            x = jnp.bitwise_xor(iota, sh)
            af = a + _shuf(a, x)
            bf = b + _shuf(b, x)
            nxt.append(jnp.where((iota & sh) != 0, bf, af))
        vecs = nxt
    return vecs[0]


def _gat_edges_sc(XL, XR, srcp, degs, meta, att06, att04, bias, *, oc, relu):
    ocn = oc // 16

    def body(xl_hbm, xr_hbm, src_hbm, deg_hbm, meta_hbm, a06_hbm, a04_hbm, b_hbm,
             out_hbm,
             mbuf, dbuf, xbuf, abuf, a06buf, a04buf, bbuf, rbuf, sbuf,
             sem0, sem1):
        wid = lax.axis_index("c") * 16 + lax.axis_index("s")
        iota = lax.iota(_I32, 16)

        pltpu.sync_copy(meta_hbm.at[wid], mbuf)
        pltpu.sync_copy(deg_hbm.at[pl.ds(wid * NPB, NPB + 16)], dbuf)
        pltpu.sync_copy(a06_hbm, a06buf)
        pltpu.sync_copy(a04_hbm, a04buf)
        pltpu.sync_copy(b_hbm, bbuf)

        mvec = mbuf[...]
        e_lo = mvec[0]
        ncnt = mvec[1]

        def node_body(i, e_pos):
            node = wid * NPB + i
            deg = dbuf[pl.ds(i, 16)][0]
            pltpu.sync_copy(xr_hbm.at[node], xbuf)

            def zero_body(fc, _):
                abuf[pl.ds(fc * 16, 16)] = jnp.zeros((16,), _F32)
                return 0
            lax.fori_loop(0, ocn, zero_body, 0)

            nchunks = (deg + 15) // 16

            def chunk_body(c, carry):
                m16, d16 = carry
                e = e_pos + c * 16
                pltpu.async_copy(src_hbm.at[e + iota], sbuf, sem0).wait()
                pltpu.async_copy(xl_hbm.at[sbuf], rbuf, sem1).wait()

                rem = deg - c * 16
                mask = iota < rem

                def fc_body(fc, ps):
                    base = fc * 16
                    xr_c = xbuf[pl.ds(base, 16)]
                    a06 = a06buf[pl.ds(base, 16)]
                    a04 = a04buf[pl.ds(base, 16)]
                    out = []
                    for j in range(16):
                        s = rbuf[j, pl.ds(base, 16)] + xr_c
                        out.append(ps[j] + a06 * s + a04 * jnp.abs(s))
                    return tuple(out)

                ps0 = tuple(jnp.zeros((16,), _F32) for _ in range(16))
                ps = lax.fori_loop(0, ocn, fc_body, ps0)
                alpha16 = _lane_sums(list(ps), iota)

                neg = jnp.float32(-3e38)
                am = jnp.where(mask, alpha16, jnp.full((16,), neg))
                cmax = _allreduce(am, jnp.maximum, iota)
                mnew = jnp.maximum(m16, cmax)
                scale = jnp.exp(m16 - mnew)
                w16 = jnp.where(mask, jnp.exp(alpha16 - mnew), jnp.zeros((16,), _F32))
                d16n = d16 * scale + w16

                wsp = [_shuf(w16, jnp.full((16,), j, _I32)) for j in range(16)]

                def agg_body(fc, _):
                    base = fc * 16
                    a = abuf[pl.ds(base, 16)] * scale
                    for j in range(16):
                        a = a + wsp[j] * rbuf[j, pl.ds(base, 16)]
                    abuf[pl.ds(base, 16)] = a
                    return 0
                lax.fori_loop(0, ocn, agg_body, 0)
                return mnew, d16n

            m0 = jnp.full((16,), -3e38, _F32)
            d0 = jnp.zeros((16,), _F32)
            _, d16 = lax.fori_loop(0, nchunks, chunk_body, (m0, d0))

            dinv = 1.0 / _allreduce(d16, lambda a, b: a + b, iota)

            def fin_body(fc, _):
                base = fc * 16
                o = abuf[pl.ds(base, 16)] * dinv + bbuf[pl.ds(base, 16)]
                if relu:
                    o = jnp.maximum(o, 0.0)
                abuf[pl.ds(base, 16)] = o
                return 0
            lax.fori_loop(0, ocn, fin_body, 0)

            pltpu.sync_copy(abuf, out_hbm.at[node])
            return e_pos + deg

        lax.fori_loop(0, ncnt, node_body, e_lo)

    mesh = plsc.VectorSubcoreMesh(core_axis_name="c", subcore_axis_name="s")
    f = pl.kernel(
        body,
        out_type=jax.ShapeDtypeStruct((N_NODES, oc), _F32),
        mesh=mesh,
        scratch_types=[
            pltpu.VMEM((16,), _I32),
            pltpu.VMEM((NPB + 16,), _I32),
            pltpu.VMEM((oc,), _F32),
            pltpu.VMEM((oc,), _F32),
            pltpu.VMEM((oc,), _F32),
            pltpu.VMEM((oc,), _F32),
            pltpu.VMEM((oc,), _F32),
            pltpu.VMEM((16, oc), _F32),
            pltpu.VMEM((16,), _I32),
            pltpu.SemaphoreType.DMA,
            pltpu.SemaphoreType.DMA,
        ],
    )(XL, XR, srcp, degs, meta, att06, att04, bias)
    return f


def kernel(x, emb, edge_index, gW, gb, cW, cb, Wl1, bl1, Wr1, br1, att1, bias1, Wl2, bl2, Wr2, br2, att2, bias2, Wl3, bl3, Wr3, br3, att3, bias3, Wl4, bl4, Wr4, br4, att4, bias4):
    N = N_NODES
    loop = jnp.arange(N, dtype=edge_index.dtype)
    src = jnp.concatenate([edge_index[0], loop])
    dst = jnp.concatenate([edge_index[1], loop])
    dst_s, src_s = lax.sort([dst, src], num_keys=1)
    offs = jnp.searchsorted(dst_s, jnp.arange(N + 1, dtype=_I32)).astype(_I32)
    degs = offs[1:] - offs[:-1]
    degs_pad = jnp.concatenate(
        [degs, jnp.zeros((NW * NPB + 16 - N,), _I32)])
    tile_nlo = jnp.arange(NW, dtype=_I32) * NPB
    e_lo = offs[tile_nlo]
    ncnt = jnp.clip(N - tile_nlo, 0, NPB)
    meta = jnp.zeros((NW, 16), _I32).at[:, 0].set(e_lo).at[:, 1].set(ncnt)
    src_pad = jnp.concatenate([src_s, jnp.zeros((16,), _I32)])

    fused = _fused_features(x, emb, gW, gb, cW, cb)

    h = fused
    outs = []
    layers = [
        (Wl1, bl1, Wr1, br1, att1, bias1, True),
        (Wl2, bl2, Wr2, br2, att2, bias2, True),
        (Wl3, bl3, Wr3, br3, att3, bias3, True),
        (Wl4, bl4, Wr4, br4, att4, bias4, False),
    ]
    for (Wl, bl, Wr, br, att, bias, relu) in layers:
        XL, XR = _project(h, Wl, bl, Wr, br)
        oc = Wl.shape[1]
        f = _gat_edges_sc(XL, XR, src_pad, degs_pad, meta,
                          0.6 * att, 0.4 * att, bias, oc=oc, relu=relu)
        outs.append(f)
        h = f
    return jnp.concatenate([outs[0], outs[1], outs[3]], axis=1)


# speedup vs baseline: 3.7842x; 1.2052x over previous
import functools

import jax
import jax.numpy as jnp
from jax import lax
from jax.experimental import pallas as pl
from jax.experimental.pallas import tpu as pltpu
from jax.experimental.pallas import tpu_sc as plsc

N_NODES = 10000
NPB = 320
NW = 32
_F32 = jnp.float32
_I32 = jnp.int32


def _fuse_body(xT, embT, gWT, gb, cWT, cb, out):
    xg = jnp.maximum(jnp.dot(xT[...], gWT[...], preferred_element_type=_F32) + gb[...], 0.0)
    xe = jnp.maximum(jnp.dot(embT[...], cWT[...], preferred_element_type=_F32) + cb[...], 0.0)
    out[...] = jnp.concatenate([xg, xe], axis=1)


def _fused_features(x, emb, gW, gb, cW, cb):
    return pl.pallas_call(
        _fuse_body,
        out_shape=jax.ShapeDtypeStruct((N_NODES, 128), _F32),
    )(x[0].T, emb[0].T, gW.T, gb[None, :], cW.T, cb[None, :])


def _proj_body(h, Wl, bl, Wr, br, xl, xr):
    hb = h[...]
    xl[...] = jnp.dot(hb, Wl[...], preferred_element_type=_F32) + bl[...]
    xr[...] = jnp.dot(hb, Wr[...], preferred_element_type=_F32) + br[...]


def _project(h, Wl, bl, Wr, br):
    n, ic = h.shape
    oc = Wl.shape[1]
    bn = 1000
    grid = n // bn
    return pl.pallas_call(
        _proj_body,
        grid=(grid,),
        in_specs=[
            pl.BlockSpec((bn, ic), lambda i: (i, 0)),
            pl.BlockSpec((ic, oc), lambda i: (0, 0)),
            pl.BlockSpec((1, oc), lambda i: (0, 0)),
            pl.BlockSpec((ic, oc), lambda i: (0, 0)),
            pl.BlockSpec((1, oc), lambda i: (0, 0)),
        ],
        out_specs=[
            pl.BlockSpec((bn, oc), lambda i: (i, 0)),
            pl.BlockSpec((bn, oc), lambda i: (i, 0)),
        ],
        out_shape=[
            jax.ShapeDtypeStruct((n, oc), _F32),
            jax.ShapeDtypeStruct((n, oc), _F32),
        ],
    )(h, Wl, bl[None, :], Wr, br[None, :])


def _shuf(v, idx):
    return v.at[idx].get(mode="promise_in_bounds")


def _allreduce(v, op, iota):
    for sh in (8, 4, 2, 1):
        v = op(v, _shuf(v, jnp.bitwise_xor(iota, sh)))
    return v


def _lane_sums(vecs, iota):
    for sh in (1, 2, 4, 8):
        nxt = []
        for i in range(0, len(vecs), 2):
            a, b = vecs[i], vecs[i + 1]
            x = jnp.bitwise_xor(iota, sh)
            af = a + _shuf(a, x)
            bf = b + _shuf(b, x)
            nxt.append(jnp.where((iota & sh) != 0, bf, af))
        vecs = nxt
    return vecs[0]


def _gat_edges_sc(XL, XR, srcp, degs, meta, att06, att04, bias, *, oc, relu):
    ocn = oc // 16

    def body(xl_hbm, xr_hbm, src_hbm, deg_hbm, meta_hbm, a06_hbm, a04_hbm, b_hbm,
             out_hbm,
             mbuf, dbuf, xbuf, abuf, a06buf, a04buf, bbuf,
             rbuf0, rbuf1, sbuf0, sbuf1,
             ssem0, ssem1, rsem0, rsem1, xsem):
        wid = lax.axis_index("c") * 16 + lax.axis_index("s")
        iota = lax.iota(_I32, 16)

        pltpu.sync_copy(meta_hbm.at[wid], mbuf)
        pltpu.sync_copy(deg_hbm.at[pl.ds(wid * NPB, NPB + 16)], dbuf)
        pltpu.sync_copy(a06_hbm, a06buf)
        pltpu.sync_copy(a04_hbm, a04buf)
        pltpu.sync_copy(b_hbm, bbuf)

        mvec = mbuf[...]
        e_lo = mvec[0]
        ncnt = mvec[1]

        def rz_body(fc, _):
            z = jnp.zeros((16,), _F32)
            for j in range(16):
                rbuf0[j, pl.ds(fc * 16, 16)] = z
                rbuf1[j, pl.ds(fc * 16, 16)] = z
            return 0
        lax.fori_loop(0, ocn, rz_body, 0)

        sb = (sbuf0, sbuf1)
        rb = (rbuf0, rbuf1)
        ssem = (ssem0, ssem1)
        rsem = (rsem0, rsem1)

        def node_body(i, e_pos):
            node = wid * NPB + i
            deg = dbuf[pl.ds(i, 16)][0]
            nchunks = (deg + 15) // 16

            pltpu.async_copy(xr_hbm.at[node], xbuf, xsem)
            pltpu.async_copy(src_hbm.at[e_pos + iota], sbuf0, ssem0)

            def zero_body(fc, _):
                abuf[pl.ds(fc * 16, 16)] = jnp.zeros((16,), _F32)
                return 0
            lax.fori_loop(0, ocn, zero_body, 0)

            pltpu.make_async_copy(src_hbm.at[e_pos + iota], sbuf0, ssem0).wait()
            pltpu.async_copy(xl_hbm.at[sbuf0], rbuf0, rsem0)

            @pl.when(nchunks > 1)
            def _():
                pltpu.async_copy(src_hbm.at[e_pos + 16 + iota], sbuf1, ssem1)

            pltpu.make_async_copy(xr_hbm.at[node], xbuf, xsem).wait()

            def compute_chunk(c, m16, d16, rbc):
                rem = deg - c * 16
                mask = iota < rem

                def fc_body(fc, ps):
                    base = fc * 16
                    xr_c = xbuf[pl.ds(base, 16)]
                    a06 = a06buf[pl.ds(base, 16)]
                    a04 = a04buf[pl.ds(base, 16)]
                    out = []
                    for j in range(16):
                        s = rbc[j, pl.ds(base, 16)] + xr_c
                        out.append(ps[j] + a06 * s + a04 * jnp.abs(s))
                    return tuple(out)

                ps0 = tuple(jnp.zeros((16,), _F32) for _ in range(16))
                ps = lax.fori_loop(0, ocn, fc_body, ps0)
                alpha16 = _lane_sums(list(ps), iota)

                neg = jnp.float32(-3e38)
                am = jnp.where(mask, alpha16, jnp.full((16,), neg))
                cmax = _allreduce(am, jnp.maximum, iota)
                mnew = jnp.maximum(m16, cmax)
                scale = jnp.exp(m16 - mnew)
                w16 = jnp.where(mask, jnp.exp(alpha16 - mnew), jnp.zeros((16,), _F32))
                d16n = d16 * scale + w16

                wsp = [_shuf(w16, jnp.full((16,), j, _I32)) for j in range(16)]

                def agg_body(fc, _):
                    base = fc * 16
                    a = abuf[pl.ds(base, 16)] * scale
                    for j in range(16):
                        a = a + wsp[j] * rbc[j, pl.ds(base, 16)]
                    abuf[pl.ds(base, 16)] = a
                    return 0
                lax.fori_loop(0, ocn, agg_body, 0)
                return mnew, d16n

            def pair_body(p, carry):
                m16, d16 = carry
                for b in range(2):
                    c = 2 * p + b
                    pn = (b + 1) % 2
                    @pl.when(c + 1 < nchunks)
                    def _():
                        e1 = e_pos + (c + 1) * 16
                        pltpu.make_async_copy(
                            src_hbm.at[e1 + iota], sb[pn], ssem[pn]).wait()
                        pltpu.async_copy(xl_hbm.at[sb[pn]], rb[pn], rsem[pn])

                    @pl.when(c < nchunks)
                    def _():
                        pltpu.make_async_copy(
                            xl_hbm.at[sb[b]], rb[b], rsem[b]).wait()

                    @pl.when(c + 2 < nchunks)
                    def _():
                        e2 = e_pos + (c + 2) * 16
                        pltpu.async_copy(src_hbm.at[e2 + iota], sb[b], ssem[b])

                    m16, d16 = compute_chunk(c, m16, d16, rb[b])
                return m16, d16

            m0 = jnp.full((16,), -3e38, _F32)
            d0 = jnp.zeros((16,), _F32)
            npairs = (nchunks + 1) // 2
            _, d16 = lax.fori_loop(0, npairs, pair_body, (m0, d0))

            dinv = 1.0 / _allreduce(d16, lambda a, b: a + b, iota)

            def fin_body(fc, _):
                base = fc * 16
                o = abuf[pl.ds(base, 16)] * dinv + bbuf[pl.ds(base, 16)]
                if relu:
                    o = jnp.maximum(o, 0.0)
                abuf[pl.ds(base, 16)] = o
                return 0
            lax.fori_loop(0, ocn, fin_body, 0)

            pltpu.sync_copy(abuf, out_hbm.at[node])
            return e_pos + deg

        lax.fori_loop(0, ncnt, node_body, e_lo)

    mesh = plsc.VectorSubcoreMesh(core_axis_name="c", subcore_axis_name="s")
    f = pl.kernel(
        body,
        out_type=jax.ShapeDtypeStruct((N_NODES, oc), _F32),
        mesh=mesh,
        scratch_types=[
            pltpu.VMEM((16,), _I32),
            pltpu.VMEM((NPB + 16,), _I32),
            pltpu.VMEM((oc,), _F32),
            pltpu.VMEM((oc,), _F32),
            pltpu.VMEM((oc,), _F32),
            pltpu.VMEM((oc,), _F32),
            pltpu.VMEM((oc,), _F32),
            pltpu.VMEM((16, oc), _F32),
            pltpu.VMEM((16, oc), _F32),
            pltpu.VMEM((16,), _I32),
            pltpu.VMEM((16,), _I32),
            pltpu.SemaphoreType.DMA,
            pltpu.SemaphoreType.DMA,
            pltpu.SemaphoreType.DMA,
            pltpu.SemaphoreType.DMA,
            pltpu.SemaphoreType.DMA,
        ],
    )(XL, XR, srcp, degs, meta, att06, att04, bias)
    return f


def kernel(x, emb, edge_index, gW, gb, cW, cb, Wl1, bl1, Wr1, br1, att1, bias1, Wl2, bl2, Wr2, br2, att2, bias2, Wl3, bl3, Wr3, br3, att3, bias3, Wl4, bl4, Wr4, br4, att4, bias4):
    N = N_NODES
    loop = jnp.arange(N, dtype=edge_index.dtype)
    src = jnp.concatenate([edge_index[0], loop])
    dst = jnp.concatenate([edge_index[1], loop])
    dst_s, src_s = lax.sort([dst, src], num_keys=1)
    offs = jnp.searchsorted(dst_s, jnp.arange(N + 1, dtype=_I32)).astype(_I32)
    degs = offs[1:] - offs[:-1]
    degs_pad = jnp.concatenate(
        [degs, jnp.zeros((NW * NPB + 16 - N,), _I32)])
    tile_nlo = jnp.arange(NW, dtype=_I32) * NPB
    e_lo = offs[tile_nlo]
    ncnt = jnp.clip(N - tile_nlo, 0, NPB)
    meta = jnp.zeros((NW, 16), _I32).at[:, 0].set(e_lo).at[:, 1].set(ncnt)
    src_pad = jnp.concatenate([src_s, jnp.zeros((16,), _I32)])

    fused = _fused_features(x, emb, gW, gb, cW, cb)

    h = fused
    outs = []
    layers = [
        (Wl1, bl1, Wr1, br1, att1, bias1, True),
        (Wl2, bl2, Wr2, br2, att2, bias2, True),
        (Wl3, bl3, Wr3, br3, att3, bias3, True),
        (Wl4, bl4, Wr4, br4, att4, bias4, False),
    ]
    for (Wl, bl, Wr, br, att, bias, relu) in layers:
        XL, XR = _project(h, Wl, bl, Wr, br)
        oc = Wl.shape[1]
        f = _gat_edges_sc(XL, XR, src_pad, degs_pad, meta,
                          0.6 * att, 0.4 * att, bias, oc=oc, relu=relu)
        outs.append(f)
        h = f
    return jnp.concatenate([outs[0], outs[1], outs[3]], axis=1)


# speedup vs baseline: 3.9760x; 1.0507x over previous
import functools

import jax
import jax.numpy as jnp
from jax import lax
from jax.experimental import pallas as pl
from jax.experimental.pallas import tpu as pltpu
from jax.experimental.pallas import tpu_sc as plsc

N_NODES = 10000
NPB = 320
NW = 32
_F32 = jnp.float32
_I32 = jnp.int32


def _fuse_body(xT, embT, gWT, gb, cWT, cb, out):
    xg = jnp.maximum(jnp.dot(xT[...], gWT[...], preferred_element_type=_F32) + gb[...], 0.0)
    xe = jnp.maximum(jnp.dot(embT[...], cWT[...], preferred_element_type=_F32) + cb[...], 0.0)
    out[...] = jnp.concatenate([xg, xe], axis=1)


def _fused_features(x, emb, gW, gb, cW, cb):
    return pl.pallas_call(
        _fuse_body,
        out_shape=jax.ShapeDtypeStruct((N_NODES, 128), _F32),
    )(x[0].T, emb[0].T, gW.T, gb[None, :], cW.T, cb[None, :])


def _proj_body(h, Wl, bl, Wr, br, xl, xr):
    hb = h[...]
    xl[...] = jnp.dot(hb, Wl[...], preferred_element_type=_F32) + bl[...]
    xr[...] = jnp.dot(hb, Wr[...], preferred_element_type=_F32) + br[...]


def _project(h, Wl, bl, Wr, br):
    n, ic = h.shape
    oc = Wl.shape[1]
    bn = 1000
    grid = n // bn
    return pl.pallas_call(
        _proj_body,
        grid=(grid,),
        in_specs=[
            pl.BlockSpec((bn, ic), lambda i: (i, 0)),
            pl.BlockSpec((ic, oc), lambda i: (0, 0)),
            pl.BlockSpec((1, oc), lambda i: (0, 0)),
            pl.BlockSpec((ic, oc), lambda i: (0, 0)),
            pl.BlockSpec((1, oc), lambda i: (0, 0)),
        ],
        out_specs=[
            pl.BlockSpec((bn, oc), lambda i: (i, 0)),
            pl.BlockSpec((bn, oc), lambda i: (i, 0)),
        ],
        out_shape=[
            jax.ShapeDtypeStruct((n, oc), _F32),
            jax.ShapeDtypeStruct((n, oc), _F32),
        ],
    )(h, Wl, bl[None, :], Wr, br[None, :])


def _shuf(v, idx):
    return v.at[idx].get(mode="promise_in_bounds")


def _allreduce(v, op, iota):
    for sh in (8, 4, 2, 1):
        v = op(v, _shuf(v, jnp.bitwise_xor(iota, sh)))
    return v


def _lane_sums(vecs, iota):
    for sh in (1, 2, 4, 8):
        nxt = []
        for i in range(0, len(vecs), 2):
            a, b = vecs[i], vecs[i + 1]
            x = jnp.bitwise_xor(iota, sh)
            af = a + _shuf(a, x)
            bf = b + _shuf(b, x)
            nxt.append(jnp.where((iota & sh) != 0, bf, af))
        vecs = nxt
    return vecs[0]


def _gat_edges_sc(XL, XR, srcp, degs, meta, att06, att04, bias, *, oc, relu):
    ocn = oc // 16

    def body(xl_hbm, xr_hbm, src_hbm, deg_hbm, meta_hbm, a06_hbm, a04_hbm, b_hbm,
             out_hbm,
             mbuf, dbuf, xbuf, abuf, obuf, a06buf, a04buf, bbuf,
             rbuf0, rbuf1, sbuf0, sbuf1,
             ssem0, ssem1, rsem0, rsem1, xsem, osem):
        wid = lax.axis_index("c") * 16 + lax.axis_index("s")
        iota = lax.iota(_I32, 16)

        pltpu.sync_copy(meta_hbm.at[wid], mbuf)
        pltpu.sync_copy(deg_hbm.at[pl.ds(wid * NPB, NPB + 16)], dbuf)
        pltpu.sync_copy(a06_hbm, a06buf)
        pltpu.sync_copy(a04_hbm, a04buf)
        pltpu.sync_copy(b_hbm, bbuf)

        mvec = mbuf[...]
        e_lo = mvec[0]
        ncnt = mvec[1]

        def rz_body(fc, _):
            z = jnp.zeros((16,), _F32)
            for j in range(16):
                rbuf0[j, pl.ds(fc * 16, 16)] = z
                rbuf1[j, pl.ds(fc * 16, 16)] = z
            return 0
        lax.fori_loop(0, ocn, rz_body, 0)

        sb = (sbuf0, sbuf1)
        rb = (rbuf0, rbuf1)
        ssem = (ssem0, ssem1)
        rsem = (rsem0, rsem1)

        pltpu.async_copy(xr_hbm.at[wid * NPB], xbuf, xsem)
        pltpu.async_copy(src_hbm.at[e_lo + iota], sbuf0, ssem0)
        pltpu.make_async_copy(src_hbm.at[e_lo + iota], sbuf0, ssem0).wait()
        pltpu.async_copy(xl_hbm.at[sbuf0], rbuf0, rsem0)

        def node_body(i, e_pos):
            node = wid * NPB + i
            deg = dbuf[pl.ds(i, 16)][0]
            nchunks = (deg + 15) // 16

            def zero_body(fc, _):
                abuf[pl.ds(fc * 16, 16)] = jnp.zeros((16,), _F32)
                return 0
            lax.fori_loop(0, ocn, zero_body, 0)

            @pl.when(nchunks > 1)
            def _():
                pltpu.async_copy(src_hbm.at[e_pos + 16 + iota], sbuf1, ssem1)

            pltpu.make_async_copy(xr_hbm.at[node], xbuf, xsem).wait()

            def compute_chunk(c, m16, d16, rbc):
                rem = deg - c * 16
                mask = iota < rem

                def fc_body(fc, ps):
                    base = fc * 16
                    xr_c = xbuf[pl.ds(base, 16)]
                    a06 = a06buf[pl.ds(base, 16)]
                    a04 = a04buf[pl.ds(base, 16)]
                    out = []
                    for j in range(16):
                        s = rbc[j, pl.ds(base, 16)] + xr_c
                        out.append(ps[j] + a06 * s + a04 * jnp.abs(s))
                    return tuple(out)

                ps0 = tuple(jnp.zeros((16,), _F32) for _ in range(16))
                ps = lax.fori_loop(0, ocn, fc_body, ps0)
                alpha16 = _lane_sums(list(ps), iota)

                neg = jnp.float32(-3e38)
                am = jnp.where(mask, alpha16, jnp.full((16,), neg))
                cmax = _allreduce(am, jnp.maximum, iota)
                mnew = jnp.maximum(m16, cmax)
                scale = jnp.exp(m16 - mnew)
                w16 = jnp.where(mask, jnp.exp(alpha16 - mnew), jnp.zeros((16,), _F32))
                d16n = d16 * scale + w16

                wsp = [_shuf(w16, jnp.full((16,), j, _I32)) for j in range(16)]

                def agg_body(fc, _):
                    base = fc * 16
                    a = abuf[pl.ds(base, 16)] * scale
                    for j in range(16):
                        a = a + wsp[j] * rbc[j, pl.ds(base, 16)]
                    abuf[pl.ds(base, 16)] = a
                    return 0
                lax.fori_loop(0, ocn, agg_body, 0)
                return mnew, d16n

            def pair_body(p, carry):
                m16, d16 = carry
                for b in range(2):
                    c = 2 * p + b
                    pn = (b + 1) % 2
                    @pl.when(c + 1 < nchunks)
                    def _():
                        e1 = e_pos + (c + 1) * 16
                        pltpu.make_async_copy(
                            src_hbm.at[e1 + iota], sb[pn], ssem[pn]).wait()
                        pltpu.async_copy(xl_hbm.at[sb[pn]], rb[pn], rsem[pn])

                    @pl.when(c < nchunks)
                    def _():
                        pltpu.make_async_copy(
                            xl_hbm.at[sb[b]], rb[b], rsem[b]).wait()

                    @pl.when(c + 2 < nchunks)
                    def _():
                        e2 = e_pos + (c + 2) * 16
                        pltpu.async_copy(src_hbm.at[e2 + iota], sb[b], ssem[b])

                    m16, d16 = compute_chunk(c, m16, d16, rb[b])
                return m16, d16

            m0 = jnp.full((16,), -3e38, _F32)
            d0 = jnp.zeros((16,), _F32)
            npairs = (nchunks + 1) // 2
            _, d16 = lax.fori_loop(0, npairs, pair_body, (m0, d0))

            e_next = e_pos + deg

            @pl.when(i + 1 < ncnt)
            def _():
                pltpu.async_copy(xr_hbm.at[node + 1], xbuf, xsem)
                pltpu.async_copy(src_hbm.at[e_next + iota], sbuf0, ssem0)

            @pl.when(i > 0)
            def _():
                pltpu.make_async_copy(obuf, out_hbm.at[node - 1], osem).wait()

            dinv = 1.0 / _allreduce(d16, lambda a, b: a + b, iota)

            def fin_body(fc, _):
                base = fc * 16
                o = abuf[pl.ds(base, 16)] * dinv + bbuf[pl.ds(base, 16)]
                if relu:
                    o = jnp.maximum(o, 0.0)
                obuf[pl.ds(base, 16)] = o
                return 0
            lax.fori_loop(0, ocn, fin_body, 0)

            pltpu.async_copy(obuf, out_hbm.at[node], osem)

            @pl.when(i + 1 < ncnt)
            def _():
                pltpu.make_async_copy(src_hbm.at[e_next + iota], sbuf0, ssem0).wait()
                pltpu.async_copy(xl_hbm.at[sbuf0], rbuf0, rsem0)
            return e_next

        e_end = lax.fori_loop(0, ncnt, node_body, e_lo)
        pltpu.make_async_copy(obuf, out_hbm.at[wid * NPB + ncnt - 1], osem).wait()

    mesh = plsc.VectorSubcoreMesh(core_axis_name="c", subcore_axis_name="s")
    f = pl.kernel(
        body,
        out_type=jax.ShapeDtypeStruct((N_NODES, oc), _F32),
        mesh=mesh,
        scratch_types=[
            pltpu.VMEM((16,), _I32),
            pltpu.VMEM((NPB + 16,), _I32),
            pltpu.VMEM((oc,), _F32),
            pltpu.VMEM((oc,), _F32),
            pltpu.VMEM((oc,), _F32),
            pltpu.VMEM((oc,), _F32),
            pltpu.VMEM((oc,), _F32),
            pltpu.VMEM((oc,), _F32),
            pltpu.VMEM((16, oc), _F32),
            pltpu.VMEM((16, oc), _F32),
            pltpu.VMEM((16,), _I32),
            pltpu.VMEM((16,), _I32),
            pltpu.SemaphoreType.DMA,
            pltpu.SemaphoreType.DMA,
            pltpu.SemaphoreType.DMA,
            pltpu.SemaphoreType.DMA,
            pltpu.SemaphoreType.DMA,
            pltpu.SemaphoreType.DMA,
        ],
    )(XL, XR, srcp, degs, meta, att06, att04, bias)
    return f


def kernel(x, emb, edge_index, gW, gb, cW, cb, Wl1, bl1, Wr1, br1, att1, bias1, Wl2, bl2, Wr2, br2, att2, bias2, Wl3, bl3, Wr3, br3, att3, bias3, Wl4, bl4, Wr4, br4, att4, bias4):
    N = N_NODES
    loop = jnp.arange(N, dtype=edge_index.dtype)
    src = jnp.concatenate([edge_index[0], loop])
    dst = jnp.concatenate([edge_index[1], loop])
    dst_s, src_s = lax.sort([dst, src], num_keys=1)
    offs = jnp.searchsorted(dst_s, jnp.arange(N + 1, dtype=_I32)).astype(_I32)
    degs = offs[1:] - offs[:-1]
    degs_pad = jnp.concatenate(
        [degs, jnp.zeros((NW * NPB + 16 - N,), _I32)])
    tile_nlo = jnp.arange(NW, dtype=_I32) * NPB
    e_lo = offs[tile_nlo]
    ncnt = jnp.clip(N - tile_nlo, 0, NPB)
    meta = jnp.zeros((NW, 16), _I32).at[:, 0].set(e_lo).at[:, 1].set(ncnt)
    src_pad = jnp.concatenate([src_s, jnp.zeros((16,), _I32)])

    fused = _fused_features(x, emb, gW, gb, cW, cb)

    h = fused
    outs = []
    layers = [
        (Wl1, bl1, Wr1, br1, att1, bias1, True),
        (Wl2, bl2, Wr2, br2, att2, bias2, True),
        (Wl3, bl3, Wr3, br3, att3, bias3, True),
        (Wl4, bl4, Wr4, br4, att4, bias4, False),
    ]
    for (Wl, bl, Wr, br, att, bias, relu) in layers:
        XL, XR = _project(h, Wl, bl, Wr, br)
        oc = Wl.shape[1]
        f = _gat_edges_sc(XL, XR, src_pad, degs_pad, meta,
                          0.6 * att, 0.4 * att, bias, oc=oc, relu=relu)
        outs.append(f)
        h = f
    return jnp.concatenate([outs[0], outs[1], outs[3]], axis=1)
